# baseline (speedup 1.0000x reference)
import functools

import jax
import jax.numpy as jnp
from jax import lax
from jax.experimental import pallas as pl
from jax.experimental.pallas import tpu as pltpu
from jax.experimental.pallas import tpu_sc as plsc

NUM_TOKENS = 8192
DIM = 256
B_TOTAL = 64 * 1024
NUM_CORES = 2
NUM_SUBCORES = 16
NW = NUM_CORES * NUM_SUBCORES
BPW = B_TOTAL // NW
CHUNK = 64
NCHUNK = BPW // CHUNK

_mesh = plsc.VectorSubcoreMesh(core_axis_name="c", subcore_axis_name="s")


@functools.partial(
    pl.kernel,
    mesh=_mesh,
    out_type=jax.ShapeDtypeStruct((B_TOTAL, DIM), jnp.float32),
    scratch_types=[
        pltpu.VMEM((NCHUNK, CHUNK), jnp.int32),
        pltpu.VMEM((6, CHUNK, DIM), jnp.float32),
        pltpu.SemaphoreType.DMA,
        pltpu.SemaphoreType.DMA,
    ],
)
def _embed_lookup(idx_hbm, table_hbm, out_hbm, idx_v, rows_v, gsem, osem):
    wid = lax.axis_index("s") * NUM_CORES + lax.axis_index("c")
    base = wid * BPW
    NBUF = 6

    pltpu.sync_copy(idx_hbm.at[wid], idx_v)

    gcp = [None] * NBUF
    ocp = {}
    for c in range(min(NBUF, NCHUNK)):
        gcp[c] = pltpu.async_copy(table_hbm.at[idx_v.at[c]], rows_v.at[c], gsem)
    for c in range(NCHUNK):
        buf = c % NBUF
        nxt = c + NBUF - 1
        if c >= 1 and nxt < NCHUNK:
            pbuf = (c - 1) % NBUF
            ocp.pop(c - 1).wait()
            gcp[pbuf] = pltpu.async_copy(
                table_hbm.at[idx_v.at[nxt]], rows_v.at[pbuf], gsem
            )
        gcp[buf].wait()
        ocp[c] = pltpu.async_copy(
            rows_v.at[buf], out_hbm.at[pl.ds(base + c * CHUNK, CHUNK)], osem
        )
    for c in sorted(ocp):
        ocp.pop(c).wait()


def kernel(embed_id, weight):
    idx = embed_id.reshape(NW, NCHUNK, CHUNK)
    out = _embed_lookup(idx, weight)
    return out.reshape(embed_id.shape[0], embed_id.shape[1], DIM)


# speedup vs baseline: 1.0083x; 1.0083x over previous
import functools

import jax
import jax.numpy as jnp
from jax import lax
from jax.experimental import pallas as pl
from jax.experimental.pallas import tpu as pltpu
from jax.experimental.pallas import tpu_sc as plsc

NUM_TOKENS = 8192
DIM = 256
B_TOTAL = 64 * 1024
NUM_CORES = 2
NUM_SUBCORES = 16
NW = NUM_CORES * NUM_SUBCORES
BPW = B_TOTAL // NW
CHUNK = 128
NCHUNK = BPW // CHUNK

_mesh = plsc.VectorSubcoreMesh(core_axis_name="c", subcore_axis_name="s")


@functools.partial(
    pl.kernel,
    mesh=_mesh,
    out_type=jax.ShapeDtypeStruct((B_TOTAL, DIM), jnp.float32),
    scratch_types=[
        pltpu.VMEM((NCHUNK, CHUNK), jnp.int32),
        pltpu.VMEM((3, CHUNK, DIM), jnp.float32),
        pltpu.SemaphoreType.DMA,
        pltpu.SemaphoreType.DMA,
    ],
)
def _embed_lookup(idx_hbm, table_hbm, out_hbm, idx_v, rows_v, gsem, osem):
    wid = lax.axis_index("s") * NUM_CORES + lax.axis_index("c")
    base = wid * BPW
    NBUF = 3

    HALF = NCHUNK // 2
    pltpu.sync_copy(idx_hbm.at[wid, pl.ds(0, HALF)], idx_v.at[pl.ds(0, HALF)])

    gcp = [None] * NBUF
    ocp = {}
    for c in range(min(NBUF, NCHUNK)):
        gcp[c] = pltpu.async_copy(table_hbm.at[idx_v.at[c]], rows_v.at[c], gsem)
    pltpu.sync_copy(
        idx_hbm.at[wid, pl.ds(HALF, NCHUNK - HALF)],
        idx_v.at[pl.ds(HALF, NCHUNK - HALF)],
    )
    for c in range(NCHUNK):
        buf = c % NBUF
        nxt = c + NBUF - 1
        if c >= 1 and nxt < NCHUNK:
            pbuf = (c - 1) % NBUF
            ocp.pop(c - 1).wait()
            gcp[pbuf] = pltpu.async_copy(
                table_hbm.at[idx_v.at[nxt]], rows_v.at[pbuf], gsem
            )
        gcp[buf].wait()
        ocp[c] = pltpu.async_copy(
            rows_v.at[buf], out_hbm.at[pl.ds(base + c * CHUNK, CHUNK)], osem
        )
    for c in sorted(ocp):
        ocp.pop(c).wait()


def kernel(embed_id, weight):
    idx = embed_id.reshape(NW, NCHUNK, CHUNK)
    out = _embed_lookup(idx, weight)
    return out.reshape(embed_id.shape[0], embed_id.shape[1], DIM)


# speedup vs baseline: 1.0110x; 1.0027x over previous
import functools

import jax
import jax.numpy as jnp
from jax import lax
from jax.experimental import pallas as pl
from jax.experimental.pallas import tpu as pltpu
from jax.experimental.pallas import tpu_sc as plsc

NUM_TOKENS = 8192
DIM = 256
B_TOTAL = 64 * 1024
NUM_CORES = 2
NUM_SUBCORES = 16
NW = NUM_CORES * NUM_SUBCORES
BPW = B_TOTAL // NW
CHUNK = 128
NCHUNK = BPW // CHUNK

_mesh = plsc.VectorSubcoreMesh(core_axis_name="c", subcore_axis_name="s")


@functools.partial(
    pl.kernel,
    mesh=_mesh,
    out_type=jax.ShapeDtypeStruct((B_TOTAL, DIM), jnp.float32),
    scratch_types=[
        pltpu.VMEM((NCHUNK, CHUNK), jnp.int32),
        pltpu.VMEM((3, CHUNK, DIM), jnp.float32),
        pltpu.SemaphoreType.DMA,
        pltpu.SemaphoreType.DMA,
    ],
)
def _embed_lookup(idx_hbm, table_hbm, out_hbm, idx_v, rows_v, gsem, osem):
    wid = lax.axis_index("s") * NUM_CORES + lax.axis_index("c")
    base = wid * BPW
    NBUF = 3

    HALF = NCHUNK // 2
    pltpu.sync_copy(idx_hbm.at[wid, pl.ds(0, HALF)], idx_v.at[pl.ds(0, HALF)])

    pieces = [(0, 0, 64), (0, 64, 64)]
    pieces += [(c, 0, CHUNK) for c in range(1, NCHUNK - 1)]
    pieces += [(NCHUNK - 1, 0, 64), (NCHUNK - 1, 64, 64)]
    NP = len(pieces)

    def fire_gather(p, buf):
        ch, off, n = pieces[p]
        return pltpu.async_copy(
            table_hbm.at[idx_v.at[ch, pl.ds(off, n)]],
            rows_v.at[buf, pl.ds(0, n)],
            gsem,
        )

    gcp = [None] * NBUF
    ocp = {}
    for p in range(min(NBUF, NP)):
        gcp[p] = fire_gather(p, p)
    pltpu.sync_copy(
        idx_hbm.at[wid, pl.ds(HALF, NCHUNK - HALF)],
        idx_v.at[pl.ds(HALF, NCHUNK - HALF)],
    )
    for p in range(NP):
        buf = p % NBUF
        nxt = p + NBUF - 1
        if p >= 1 and nxt < NP:
            pbuf = (p - 1) % NBUF
            ocp.pop(p - 1).wait()
            gcp[pbuf] = fire_gather(nxt, pbuf)
        gcp[buf].wait()
        ch, off, n = pieces[p]
        ocp[p] = pltpu.async_copy(
            rows_v.at[buf, pl.ds(0, n)],
            out_hbm.at[pl.ds(base + ch * CHUNK + off, n)],
            osem,
        )
    for p in sorted(ocp):
        ocp.pop(p).wait()


def kernel(embed_id, weight):
    idx = embed_id.reshape(NW, NCHUNK, CHUNK)
    out = _embed_lookup(idx, weight)
    return out.reshape(embed_id.shape[0], embed_id.shape[1], DIM)
